# Initial kernel scaffold; baseline (speedup 1.0000x reference)
import functools

import jax
import jax.numpy as jnp
from jax import lax
from jax.experimental import pallas as pl
from jax.experimental.pallas import tpu as pltpu
from jax.experimental.pallas import tpu_sc as plsc

NUM_TX = 4
NUM_STREAMS = 2
NUM_SYM = 14
FFT = 2048
PILOT_SYMS = (2, 11)
PILOT_STRIDE = 4
PILOTS_PER_SYM = FFT // PILOT_STRIDE
PILOTS_PER_STREAM = PILOTS_PER_SYM * 2
DATA_PER_STREAM = NUM_SYM * FFT - PILOTS_PER_STREAM
GRID_PER_STREAM = NUM_SYM * FFT

LANES = 16


def _sc_grid_mapper(flat_in, flat_pilots, *, rows, rows_per_worker):
    n_pilots = NUM_TX * NUM_STREAMS * PILOTS_PER_STREAM
    pil_base = DATA_PER_STREAM
    mesh = plsc.VectorSubcoreMesh(core_axis_name="c", subcore_axis_name="s")
    info = plsc.get_sparse_core_info()
    nc = info.num_cores

    dense_chunks = []
    pilot_rows = []
    src = 0
    for s in range(NUM_SYM):
        if s in PILOT_SYMS:
            pilot_rows.append((PILOT_SYMS.index(s), src, s * FFT))
            src += FFT - PILOTS_PER_SYM
        else:
            if dense_chunks and dense_chunks[-1][0] + dense_chunks[-1][2] == src:
                so, do, ln = dense_chunks[-1]
                dense_chunks[-1] = (so, do, ln + FFT)
            else:
                dense_chunks.append((src, s * FFT, FFT))
            src += FFT

    @functools.partial(
        pl.kernel,
        mesh=mesh,
        out_type=jax.ShapeDtypeStruct((rows * GRID_PER_STREAM,), jnp.float32),
        scratch_types=[
            pltpu.VMEM((DATA_PER_STREAM + n_pilots,), jnp.float32),
            pltpu.VMEM((FFT,), jnp.float32),
            pltpu.VMEM((FFT,), jnp.int32),
        ],
    )
    def grid_mapper(in_hbm, pil_hbm, out_hbm, big_v, row_v, idx_v):
        wid = lax.axis_index("s") * nc + lax.axis_index("c")
        lanes = lax.iota(jnp.int32, LANES)
        is_pilot_lane = (lanes & (PILOT_STRIDE - 1)) == 0

        pltpu.sync_copy(pil_hbm, big_v.at[pl.ds(pil_base, n_pilots)])

        def precomp(i, _):
            k = lanes + i * LANES
            v = jnp.where(is_pilot_lane, k >> 2, k - (k >> 2) - 1)
            idx_v[pl.ds(i * LANES, LANES)] = v
            return 0

        lax.fori_loop(0, FFT // LANES, precomp, 0)

        def per_row(t, _):
            r = wid * rows_per_worker + t
            txs = r & (NUM_TX * NUM_STREAMS - 1)
            ibase = r * DATA_PER_STREAM
            obase = r * GRID_PER_STREAM

            pltpu.sync_copy(in_hbm.at[pl.ds(ibase, DATA_PER_STREAM)],
                            big_v.at[pl.ds(0, DATA_PER_STREAM)])
            for so, do, ln in dense_chunks:
                pltpu.sync_copy(big_v.at[pl.ds(so, ln)],
                                out_hbm.at[pl.ds(obase + do, ln)])

            for psym, doff, ooff in pilot_rows:
                poff = pil_base + txs * PILOTS_PER_STREAM + psym * PILOTS_PER_SYM
                offv = jnp.where(is_pilot_lane,
                                 jnp.full((LANES,), poff, jnp.int32),
                                 jnp.full((LANES,), doff, jnp.int32))

                def gather_group(i, _, offv=offv):
                    iv = idx_v[pl.ds(i * LANES, LANES)] + offv
                    row_v[pl.ds(i * LANES, LANES)] = plsc.load_gather(big_v, [iv])
                    return 0

                lax.fori_loop(0, FFT // LANES, gather_group, 0)
                pltpu.sync_copy(row_v, out_hbm.at[pl.ds(obase + ooff, FFT)])
            return 0

        lax.fori_loop(0, rows_per_worker, per_row, 0)

    return grid_mapper(flat_in, flat_pilots)


def kernel(inputs, pilots):
    batch, num_tx, num_streams, _ = inputs.shape
    rows = batch * num_tx * num_streams
    out = _sc_grid_mapper(
        inputs.reshape(-1), pilots.reshape(-1),
        rows=rows, rows_per_worker=rows // 32)
    return out.reshape(batch, num_tx, num_streams, NUM_SYM, FFT)


# speedup vs baseline: 8.4438x; 8.4438x over previous
import functools

import jax
import jax.numpy as jnp
from jax import lax
from jax.experimental import pallas as pl
from jax.experimental.pallas import tpu as pltpu
from jax.experimental.pallas import tpu_sc as plsc

NUM_TX = 4
NUM_STREAMS = 2
NUM_SYM = 14
FFT = 2048
PILOT_SYMS = (2, 11)
PILOT_STRIDE = 4
PILOTS_PER_SYM = FFT // PILOT_STRIDE
PILOTS_PER_STREAM = PILOTS_PER_SYM * 2
DATA_PER_STREAM = NUM_SYM * FFT - PILOTS_PER_STREAM
GRID_PER_STREAM = NUM_SYM * FFT

LANES = 16


def _sc_grid_mapper(flat_in, flat_pilots, *, rows, rows_per_worker):
    n_pilots = NUM_TX * NUM_STREAMS * PILOTS_PER_STREAM
    pil_base = DATA_PER_STREAM
    mesh = plsc.VectorSubcoreMesh(core_axis_name="c", subcore_axis_name="s")
    info = plsc.get_sparse_core_info()
    nc = info.num_cores

    dense_chunks = []
    pilot_rows = []
    src = 0
    for s in range(NUM_SYM):
        if s in PILOT_SYMS:
            pilot_rows.append((PILOT_SYMS.index(s), src, s * FFT))
            src += FFT - PILOTS_PER_SYM
        else:
            if dense_chunks and dense_chunks[-1][0] + dense_chunks[-1][2] == src:
                so, do, ln = dense_chunks[-1]
                dense_chunks[-1] = (so, do, ln + FFT)
            else:
                dense_chunks.append((src, s * FFT, FFT))
            src += FFT

    @functools.partial(
        pl.kernel,
        mesh=mesh,
        out_type=jax.ShapeDtypeStruct((rows * GRID_PER_STREAM,), jnp.float32),
        scratch_types=[
            pltpu.VMEM((DATA_PER_STREAM + n_pilots,), jnp.float32),
            pltpu.VMEM((FFT,), jnp.float32),
            pltpu.VMEM((FFT,), jnp.int32),
        ],
        compiler_params=pltpu.CompilerParams(needs_layout_passes=False),
    )
    def grid_mapper(in_hbm, pil_hbm, out_hbm, big_v, row_v, idx_v):
        wid = lax.axis_index("s") * nc + lax.axis_index("c")
        lanes = lax.iota(jnp.int32, LANES)
        is_pilot_lane = (lanes & (PILOT_STRIDE - 1)) == 0

        pltpu.sync_copy(pil_hbm, big_v.at[pl.ds(pil_base, n_pilots)])

        def precomp(i, _):
            k = lanes + i * LANES
            v = jnp.where(is_pilot_lane, k >> 2, k - (k >> 2) - 1)
            idx_v[pl.ds(i * LANES, LANES)] = v
            return 0

        lax.fori_loop(0, FFT // LANES, precomp, 0)

        def per_row(t, _):
            r = wid * rows_per_worker + t
            txs = r & (NUM_TX * NUM_STREAMS - 1)
            ibase = r * DATA_PER_STREAM
            obase = r * GRID_PER_STREAM

            pltpu.sync_copy(in_hbm.at[pl.ds(ibase, DATA_PER_STREAM)],
                            big_v.at[pl.ds(0, DATA_PER_STREAM)])
            for so, do, ln in dense_chunks:
                pltpu.sync_copy(big_v.at[pl.ds(so, ln)],
                                out_hbm.at[pl.ds(obase + do, ln)])

            for psym, doff, ooff in pilot_rows:
                poff = pil_base + txs * PILOTS_PER_STREAM + psym * PILOTS_PER_SYM
                offv = jnp.where(is_pilot_lane,
                                 jnp.full((LANES,), poff, jnp.int32),
                                 jnp.full((LANES,), doff, jnp.int32))

                def gather_group(i, _, offv=offv):
                    iv = idx_v[pl.ds(i * LANES, LANES)] + offv
                    row_v[pl.ds(i * LANES, LANES)] = plsc.load_gather(big_v, [iv])
                    return 0

                lax.fori_loop(0, FFT // LANES, gather_group, 0)
                pltpu.sync_copy(row_v, out_hbm.at[pl.ds(obase + ooff, FFT)])
            return 0

        lax.fori_loop(0, rows_per_worker, per_row, 0)

    return grid_mapper(flat_in, flat_pilots)


def kernel(inputs, pilots):
    batch, num_tx, num_streams, _ = inputs.shape
    rows = batch * num_tx * num_streams
    out = _sc_grid_mapper(
        inputs.reshape(-1), pilots.reshape(-1),
        rows=rows, rows_per_worker=rows // 32)
    return out.reshape(batch, num_tx, num_streams, NUM_SYM, FFT)


# speedup vs baseline: 9.3703x; 1.1097x over previous
import functools

import jax
import jax.numpy as jnp
from jax import lax
from jax.experimental import pallas as pl
from jax.experimental.pallas import tpu as pltpu
from jax.experimental.pallas import tpu_sc as plsc

NUM_TX = 4
NUM_STREAMS = 2
NUM_SYM = 14
FFT = 2048
PILOT_SYMS = (2, 11)
PILOT_STRIDE = 4
PILOTS_PER_SYM = FFT // PILOT_STRIDE
PILOTS_PER_STREAM = PILOTS_PER_SYM * 2
DATA_PER_STREAM = NUM_SYM * FFT - PILOTS_PER_STREAM
GRID_PER_STREAM = NUM_SYM * FFT

LANES = 16
N_ROW_BUFS = 4


def _sc_grid_mapper(flat_in, flat_pilots, *, rows, rows_per_worker):
    n_pilots = NUM_TX * NUM_STREAMS * PILOTS_PER_STREAM
    pil_base = 2 * DATA_PER_STREAM
    mesh = plsc.VectorSubcoreMesh(core_axis_name="c", subcore_axis_name="s")
    info = plsc.get_sparse_core_info()
    nc = info.num_cores

    dense_chunks = []
    pilot_rows = []
    src = 0
    for s in range(NUM_SYM):
        if s in PILOT_SYMS:
            pilot_rows.append((PILOT_SYMS.index(s), src, s * FFT))
            src += FFT - PILOTS_PER_SYM
        else:
            if dense_chunks and dense_chunks[-1][0] + dense_chunks[-1][2] == src:
                so, do, ln = dense_chunks[-1]
                dense_chunks[-1] = (so, do, ln + FFT)
            else:
                dense_chunks.append((src, s * FFT, FFT))
            src += FFT

    @functools.partial(
        pl.kernel,
        mesh=mesh,
        out_type=jax.ShapeDtypeStruct((rows * GRID_PER_STREAM,), jnp.float32),
        scratch_types=[
            pltpu.VMEM((2 * DATA_PER_STREAM + n_pilots,), jnp.float32),
            pltpu.VMEM((N_ROW_BUFS * FFT,), jnp.float32),
            pltpu.VMEM((FFT,), jnp.int32),
            pltpu.SemaphoreType.DMA,
            pltpu.SemaphoreType.DMA,
            pltpu.SemaphoreType.DMA,
            pltpu.SemaphoreType.DMA,
            pltpu.SemaphoreType.DMA,
            pltpu.SemaphoreType.DMA,
            pltpu.SemaphoreType.DMA,
        ],
        compiler_params=pltpu.CompilerParams(needs_layout_passes=False),
    )
    def grid_mapper(in_hbm, pil_hbm, out_hbm, big_v, row_v, idx_v,
                    sem_in, sem_d0, sem_d1, sem_r0, sem_r1, sem_r2, sem_r3):
        dense_sems = (sem_d0, sem_d1)
        row_sems = (sem_r0, sem_r1, sem_r2, sem_r3)
        wid = lax.axis_index("s") * nc + lax.axis_index("c")
        lanes = lax.iota(jnp.int32, LANES)
        is_pilot_lane = (lanes & (PILOT_STRIDE - 1)) == 0

        pltpu.sync_copy(pil_hbm, big_v.at[pl.ds(pil_base, n_pilots)])

        def precomp(i, _):
            k = lanes + i * LANES
            v = jnp.where(is_pilot_lane, k >> 2, k - (k >> 2) - 1)
            idx_v[pl.ds(i * LANES, LANES)] = v
            return 0

        lax.fori_loop(0, FFT // LANES, precomp, 0)

        def start_in(t):
            r = wid * rows_per_worker + t
            return pltpu.async_copy(
                in_hbm.at[pl.ds(r * DATA_PER_STREAM, DATA_PER_STREAM)],
                big_v.at[pl.ds((t % 2) * DATA_PER_STREAM, DATA_PER_STREAM)],
                sem_in)

        in_descs = {0: start_in(0)}
        dense_descs = {}
        row_descs = {}
        for t in range(rows_per_worker):
            slot = t % 2
            if t - 1 in dense_descs:
                for d in dense_descs.pop(t - 1):
                    d.wait()
            if t + 1 < rows_per_worker:
                in_descs[t + 1] = start_in(t + 1)
            in_descs.pop(t).wait()

            r = wid * rows_per_worker + t
            txs = r & (NUM_TX * NUM_STREAMS - 1)
            obase = r * GRID_PER_STREAM
            sbase = slot * DATA_PER_STREAM

            dense_descs[t] = [
                pltpu.async_copy(big_v.at[pl.ds(sbase + so, ln)],
                                 out_hbm.at[pl.ds(obase + do, ln)],
                                 dense_sems[slot])
                for so, do, ln in dense_chunks]

            for j, (psym, doff, ooff) in enumerate(pilot_rows):
                rs = (len(pilot_rows) * t + j) % N_ROW_BUFS
                if (t - 2, j) in row_descs:
                    row_descs.pop((t - 2, j)).wait()
                poff = pil_base + txs * PILOTS_PER_STREAM + psym * PILOTS_PER_SYM
                offv = jnp.where(is_pilot_lane,
                                 jnp.full((LANES,), poff, jnp.int32),
                                 jnp.full((LANES,), sbase + doff, jnp.int32))

                def gather_group(i, _, offv=offv, rs=rs):
                    iv = idx_v[pl.ds(i * LANES, LANES)] + offv
                    row_v[pl.ds(rs * FFT + i * LANES, LANES)] = (
                        plsc.load_gather(big_v, [iv]))
                    return 0

                lax.fori_loop(0, FFT // LANES, gather_group, 0)
                row_descs[(t, j)] = pltpu.async_copy(
                    row_v.at[pl.ds(rs * FFT, FFT)],
                    out_hbm.at[pl.ds(obase + ooff, FFT)],
                    row_sems[rs])

        for descs in dense_descs.values():
            for d in descs:
                d.wait()
        for d in row_descs.values():
            d.wait()

    return grid_mapper(flat_in, flat_pilots)


def kernel(inputs, pilots):
    batch, num_tx, num_streams, _ = inputs.shape
    rows = batch * num_tx * num_streams
    out = _sc_grid_mapper(
        inputs.reshape(-1), pilots.reshape(-1),
        rows=rows, rows_per_worker=rows // 32)
    return out.reshape(batch, num_tx, num_streams, NUM_SYM, FFT)


# speedup vs baseline: 9.4786x; 1.0116x over previous
import functools

import jax
import jax.numpy as jnp
from jax import lax
from jax.experimental import pallas as pl
from jax.experimental.pallas import tpu as pltpu
from jax.experimental.pallas import tpu_sc as plsc

NUM_TX = 4
NUM_STREAMS = 2
NUM_SYM = 14
FFT = 2048
PILOT_SYMS = (2, 11)
PILOT_STRIDE = 4
PILOTS_PER_SYM = FFT // PILOT_STRIDE
PILOTS_PER_STREAM = PILOTS_PER_SYM * 2
DATA_PER_STREAM = NUM_SYM * FFT - PILOTS_PER_STREAM
GRID_PER_STREAM = NUM_SYM * FFT

LANES = 16
N_ROW_BUFS = 4


def _sc_grid_mapper(flat_in, flat_pilots, *, rows, rows_per_worker):
    n_pilots = NUM_TX * NUM_STREAMS * PILOTS_PER_STREAM
    pil_base = 2 * DATA_PER_STREAM
    mesh = plsc.VectorSubcoreMesh(core_axis_name="c", subcore_axis_name="s")
    info = plsc.get_sparse_core_info()
    nc = info.num_cores

    dense_chunks = []
    pilot_rows = []
    src = 0
    for s in range(NUM_SYM):
        if s in PILOT_SYMS:
            pilot_rows.append((PILOT_SYMS.index(s), src, s * FFT))
            src += FFT - PILOTS_PER_SYM
        else:
            if dense_chunks and dense_chunks[-1][0] + dense_chunks[-1][2] == src:
                so, do, ln = dense_chunks[-1]
                dense_chunks[-1] = (so, do, ln + FFT)
            else:
                dense_chunks.append((src, s * FFT, FFT))
            src += FFT

    @functools.partial(
        pl.kernel,
        mesh=mesh,
        out_type=jax.ShapeDtypeStruct((rows * GRID_PER_STREAM,), jnp.float32),
        scratch_types=[
            pltpu.VMEM((2 * DATA_PER_STREAM + n_pilots,), jnp.float32),
            pltpu.VMEM((N_ROW_BUFS * FFT,), jnp.float32),
            pltpu.SemaphoreType.DMA,
            pltpu.SemaphoreType.DMA,
            pltpu.SemaphoreType.DMA,
            pltpu.SemaphoreType.DMA,
            pltpu.SemaphoreType.DMA,
            pltpu.SemaphoreType.DMA,
            pltpu.SemaphoreType.DMA,
        ],
        compiler_params=pltpu.CompilerParams(needs_layout_passes=False),
    )
    def grid_mapper(in_hbm, pil_hbm, out_hbm, big_v, row_v,
                    sem_in, sem_d0, sem_d1, sem_r0, sem_r1, sem_r2, sem_r3):
        dense_sems = (sem_d0, sem_d1)
        row_sems = (sem_r0, sem_r1, sem_r2, sem_r3)
        wid = lax.axis_index("s") * nc + lax.axis_index("c")
        lanes = lax.iota(jnp.int32, LANES)
        is_pilot_lane = (lanes & (PILOT_STRIDE - 1)) == 0
        iv_pat0 = jnp.where(is_pilot_lane, lanes >> 2,
                            lanes - (lanes >> 2) - 1)
        iv_inc = jnp.where(is_pilot_lane,
                           jnp.full((LANES,), LANES // PILOT_STRIDE, jnp.int32),
                           jnp.full((LANES,), LANES - LANES // PILOT_STRIDE, jnp.int32))

        pltpu.sync_copy(pil_hbm, big_v.at[pl.ds(pil_base, n_pilots)])

        def start_in(t):
            r = wid * rows_per_worker + t
            return pltpu.async_copy(
                in_hbm.at[pl.ds(r * DATA_PER_STREAM, DATA_PER_STREAM)],
                big_v.at[pl.ds((t % 2) * DATA_PER_STREAM, DATA_PER_STREAM)],
                sem_in)

        in_descs = {0: start_in(0)}
        dense_descs = {}
        row_descs = {}
        for t in range(rows_per_worker):
            slot = t % 2
            if t - 1 in dense_descs:
                for d in dense_descs.pop(t - 1):
                    d.wait()
            if t + 1 < rows_per_worker:
                in_descs[t + 1] = start_in(t + 1)
            in_descs.pop(t).wait()

            r = wid * rows_per_worker + t
            txs = r & (NUM_TX * NUM_STREAMS - 1)
            obase = r * GRID_PER_STREAM
            sbase = slot * DATA_PER_STREAM

            dense_descs[t] = [
                pltpu.async_copy(big_v.at[pl.ds(sbase + so, ln)],
                                 out_hbm.at[pl.ds(obase + do, ln)],
                                 dense_sems[slot])
                for so, do, ln in dense_chunks]

            for j, (psym, doff, ooff) in enumerate(pilot_rows):
                rs = (len(pilot_rows) * t + j) % N_ROW_BUFS
                if (t - 2, j) in row_descs:
                    row_descs.pop((t - 2, j)).wait()
                poff = pil_base + txs * PILOTS_PER_STREAM + psym * PILOTS_PER_SYM
                offv = jnp.where(is_pilot_lane,
                                 jnp.full((LANES,), poff, jnp.int32),
                                 jnp.full((LANES,), sbase + doff, jnp.int32))

                @plsc.parallel_loop(0, FFT // LANES, unroll=8,
                                    carry=iv_pat0 + offv)
                def gather_group(i, iv, rs=rs):
                    row_v[pl.ds(rs * FFT + i * LANES, LANES)] = (
                        plsc.load_gather(big_v, [iv]))
                    return iv + iv_inc
                row_descs[(t, j)] = pltpu.async_copy(
                    row_v.at[pl.ds(rs * FFT, FFT)],
                    out_hbm.at[pl.ds(obase + ooff, FFT)],
                    row_sems[rs])

        for descs in dense_descs.values():
            for d in descs:
                d.wait()
        for d in row_descs.values():
            d.wait()

    return grid_mapper(flat_in, flat_pilots)


def kernel(inputs, pilots):
    batch, num_tx, num_streams, _ = inputs.shape
    rows = batch * num_tx * num_streams
    out = _sc_grid_mapper(
        inputs.reshape(-1), pilots.reshape(-1),
        rows=rows, rows_per_worker=rows // 32)
    return out.reshape(batch, num_tx, num_streams, NUM_SYM, FFT)


# speedup vs baseline: 33.5706x; 3.5417x over previous
import jax
import jax.numpy as jnp
from jax import lax
from jax.experimental import pallas as pl
from jax.experimental.pallas import tpu as pltpu
from jax.experimental.pallas import tpu_sc as plsc

NUM_TX = 4
NUM_STREAMS = 2
NUM_SYM = 14
FFT = 2048
PILOT_SYMS = (2, 11)
PILOT_STRIDE = 4
PILOTS_PER_SYM = FFT // PILOT_STRIDE
PILOTS_PER_STREAM = PILOTS_PER_SYM * 2
DATA_PER_STREAM = NUM_SYM * FFT - PILOTS_PER_STREAM
GRID_PER_STREAM = NUM_SYM * FFT

LANES = 16
BLK = 128
PAIR_IN = NUM_STREAMS * DATA_PER_STREAM
PAIR_OUT = NUM_STREAMS * GRID_PER_STREAM
SYM_OUT = NUM_STREAMS * FFT
N_PILOTS = NUM_TX * NUM_STREAMS * PILOTS_PER_STREAM
PIL_BASE = 2 * PAIR_IN
GROUPS = SYM_OUT // NUM_STREAMS // LANES


def _sc_grid_mapper(flat_in, flat_pilots, *, pairs, pairs_per_worker):
    mesh = plsc.VectorSubcoreMesh(core_axis_name="c", subcore_axis_name="s")
    info = plsc.get_sparse_core_info()
    nc = info.num_cores

    dense_chunks = []
    pilot_rows = []
    src = 0
    for s in range(NUM_SYM):
        if s in PILOT_SYMS:
            pilot_rows.append((PILOT_SYMS.index(s), src, s))
            src += FFT - PILOTS_PER_SYM
        else:
            so, do = NUM_STREAMS * src, SYM_OUT * s
            if dense_chunks and dense_chunks[-1][0] + dense_chunks[-1][2] == so:
                po, pd, ln = dense_chunks[-1]
                dense_chunks[-1] = (po, pd, ln + SYM_OUT)
            else:
                dense_chunks.append((so, do, SYM_OUT))
            src += FFT

    @pl.kernel(
        mesh=mesh,
        out_type=jax.ShapeDtypeStruct((pairs * PAIR_OUT,), jnp.float32),
        scratch_types=[
            pltpu.VMEM((PIL_BASE + N_PILOTS,), jnp.float32),
            pltpu.VMEM((2 * SYM_OUT,), jnp.float32),
            pltpu.VMEM((GROUPS * LANES,), jnp.int32),
            pltpu.SemaphoreType.DMA,
            pltpu.SemaphoreType.DMA,
            pltpu.SemaphoreType.DMA,
            pltpu.SemaphoreType.DMA,
            pltpu.SemaphoreType.DMA,
        ],
        compiler_params=pltpu.CompilerParams(needs_layout_passes=False),
    )
    def grid_mapper(in_hbm, pil_hbm, out_hbm, big_v, row_v, idx_v,
                    sem_in, sem_d0, sem_d1, sem_r0, sem_r1):
        dense_sems = (sem_d0, sem_d1)
        row_sems = (sem_r0, sem_r1)
        wid = lax.axis_index("s") * nc + lax.axis_index("c")
        lanes = lax.iota(jnp.int32, LANES)
        is_pilot_lane = (lanes & (PILOT_STRIDE - 1)) == 0

        pltpu.sync_copy(pil_hbm, big_v.at[pl.ds(PIL_BASE, N_PILOTS)])

        @plsc.parallel_loop(0, GROUPS, unroll=4)
        def build_idx(gt):
            k = gt * LANES + lanes
            h = k - (k >> 2) - 1
            kq = k >> 2
            dpat = ((h >> 7) << 8) + (h & (BLK - 1))
            ppat = ((kq >> 7) << 8) + (kq & (BLK - 1))
            idx_v[pl.ds(gt * LANES, LANES)] = jnp.where(
                is_pilot_lane, ppat, dpat)

        def start_in(u):
            q = wid * pairs_per_worker + u
            return pltpu.async_copy(
                in_hbm.at[pl.ds(q * PAIR_IN, PAIR_IN)],
                big_v.at[pl.ds((u % 2) * PAIR_IN, PAIR_IN)],
                sem_in)

        in_descs = {0: start_in(0)}
        dense_descs = {}
        row_descs = {}
        for u in range(pairs_per_worker):
            slot = u % 2
            if u - 1 in dense_descs:
                for d in dense_descs.pop(u - 1):
                    d.wait()
            if u + 1 < pairs_per_worker:
                in_descs[u + 1] = start_in(u + 1)
            in_descs.pop(u).wait()

            q = wid * pairs_per_worker + u
            tx = q & (NUM_TX - 1)
            obase = q * PAIR_OUT
            sbase = slot * PAIR_IN

            dense_descs[u] = [
                pltpu.async_copy(big_v.at[pl.ds(sbase + so, ln)],
                                 out_hbm.at[pl.ds(obase + do, ln)],
                                 dense_sems[slot])
                for so, do, ln in dense_chunks]

            for j, (psym, doff, osym) in enumerate(pilot_rows):
                rs = (len(pilot_rows) * u + j) % 2
                if (u - 1, j) in row_descs:
                    row_descs.pop((u - 1, j)).wait()
                pil_off = (PIL_BASE + tx * NUM_STREAMS * PILOTS_PER_STREAM
                           + psym * NUM_STREAMS * PILOTS_PER_SYM)
                dat_off = sbase + NUM_STREAMS * doff
                for s in range(NUM_STREAMS):
                    offv = jnp.where(
                        is_pilot_lane,
                        jnp.full((LANES,), pil_off + s * BLK, jnp.int32),
                        jnp.full((LANES,), dat_off + s * BLK, jnp.int32))
                    rbase = rs * SYM_OUT + s * BLK

                    @plsc.parallel_loop(0, GROUPS, unroll=8)
                    def gather_group(g, offv=offv, rbase=rbase):
                        iv = idx_v[pl.ds(g * LANES, LANES)] + offv
                        dest = rbase + ((g >> 3) << 8) + ((g & 7) << 4)
                        row_v[pl.ds(dest, LANES)] = (
                            plsc.load_gather(big_v, [iv]))

                row_descs[(u, j)] = pltpu.async_copy(
                    row_v.at[pl.ds(rs * SYM_OUT, SYM_OUT)],
                    out_hbm.at[pl.ds(obase + osym * SYM_OUT, SYM_OUT)],
                    row_sems[rs])

        for descs in dense_descs.values():
            for d in descs:
                d.wait()
        for d in row_descs.values():
            d.wait()

    return grid_mapper(flat_in, flat_pilots)


def kernel(inputs, pilots):
    batch, num_tx, num_streams, dps = inputs.shape
    pairs = batch * num_tx
    flat_in = inputs.reshape(
        batch, num_tx, num_streams, dps // BLK, BLK).transpose(
        0, 1, 3, 2, 4).reshape(-1)
    flat_pil = pilots.reshape(
        num_tx, num_streams, PILOTS_PER_STREAM // BLK, BLK).transpose(
        0, 2, 1, 3).reshape(-1)
    out = _sc_grid_mapper(
        flat_in, flat_pil, pairs=pairs, pairs_per_worker=pairs // 32)
    out = out.reshape(
        batch, num_tx, NUM_SYM, FFT // BLK, num_streams, BLK).transpose(
        0, 1, 4, 2, 3, 5).reshape(
        batch, num_tx, num_streams, NUM_SYM, FFT)
    return out
